# Initial kernel scaffold; baseline (speedup 1.0000x reference)
import jax
import jax.numpy as jnp
from jax.experimental import pallas as pl


def kernel(h, x, edge_index, edge_attr, batch, params):
    raise NotImplementedError("write your pallas kernel here")


# speedup vs baseline: 3.1818x; 3.1818x over previous
import functools

import jax
import jax.numpy as jnp
from jax import lax
from jax.experimental import pallas as pl
from jax.experimental.pallas import tpu as pltpu
from jax.experimental.pallas import tpu_sc as plsc

NC = 2
NS = 16
CH = 128
H = 128
BE = 1280
BN = 2000


def _silu(t):
    return t * jax.nn.sigmoid(t)


def _ln(t, g, b):
    mu = jnp.mean(t, -1, keepdims=True)
    var = jnp.mean((t - mu) ** 2, -1, keepdims=True)
    return (t - mu) * lax.rsqrt(var + 1e-5) * g + b


def _sc_gather(tab_a, tab_b, row, col, x0, x1, x2):
    n_edges = row.shape[0]
    nchunk = n_edges // CH
    per_core = nchunk // NC
    kmax = (per_core + NS - 1) // NS
    mesh = plsc.VectorSubcoreMesh(core_axis_name="c", subcore_axis_name="s")

    @functools.partial(
        pl.kernel,
        out_type=[jax.ShapeDtypeStruct((n_edges, H), jnp.float32),
                  jax.ShapeDtypeStruct((nchunk, CH), jnp.float32)],
        mesh=mesh,
        scratch_types=[
            pltpu.VMEM((CH,), jnp.int32),
            pltpu.VMEM((CH,), jnp.int32),
            pltpu.VMEM((CH, H), jnp.float32),
            pltpu.VMEM((CH,), jnp.float32),
            pltpu.VMEM((CH,), jnp.float32),
            pltpu.VMEM((CH,), jnp.float32),
            pltpu.VMEM((CH,), jnp.float32),
            pltpu.VMEM((CH,), jnp.float32),
            pltpu.VMEM((CH,), jnp.float32),
            pltpu.VMEM((CH,), jnp.float32),
        ] + [pltpu.SemaphoreType.DMA] * 8,
    )
    def k(ta, tb, row_h, col_h, x0_h, x1_h, x2_h, sg_h, dsq_h,
          idxr, idxc, buf, xr0, xr1, xr2, xc0, xc1, xc2, dsqv,
          s1, s2, s3, s4, s5, s6, s7, s8):
        c = lax.axis_index("c")
        s = lax.axis_index("s")

        def body(kk, carry):
            @pl.when(s + NS * kk < per_core)
            def _():
                t = c * per_core + s + NS * kk
                base = pl.multiple_of(t * CH, CH)
                pltpu.sync_copy(row_h.at[pl.ds(base, CH)], idxr)
                pltpu.sync_copy(col_h.at[pl.ds(base, CH)], idxc)
                ca = pltpu.async_copy(ta.at[idxr], buf, s1)
                g0 = pltpu.async_copy(x0_h.at[idxr], xr0, s2)
                g1 = pltpu.async_copy(x1_h.at[idxr], xr1, s3)
                g2 = pltpu.async_copy(x2_h.at[idxr], xr2, s4)
                g3 = pltpu.async_copy(x0_h.at[idxc], xc0, s5)
                g4 = pltpu.async_copy(x1_h.at[idxc], xc1, s6)
                g5 = pltpu.async_copy(x2_h.at[idxc], xc2, s7)
                ca.wait()
                cb = pltpu.async_copy(tb.at[idxc], buf, s8, add=True)
                g0.wait(); g1.wait(); g2.wait()
                g3.wait(); g4.wait(); g5.wait()
                for g in range(CH // 16):
                    d = pl.ds(g * 16, 16)
                    a = xr0[d] - xc0[d]
                    b = xr1[d] - xc1[d]
                    cc = xr2[d] - xc2[d]
                    dsqv[d] = a * a + b * b + cc * cc
                cb.wait()
                pltpu.sync_copy(buf, sg_h.at[pl.ds(base, CH)])
                pltpu.sync_copy(dsqv, dsq_h.at[t])
            return carry

        lax.fori_loop(0, kmax, body, 0)

    return k(tab_a, tab_b, row, col, x0, x1, x2)


def _sc_scatter(medge, row, zeros_m, qpk=None, col=None, x0=None, x1=None,
                x2=None, zeros_1=None):
    has_coord = qpk is not None
    n_edges = row.shape[0]
    n_nodes = zeros_m.shape[0]
    nchunk = n_edges // CH
    per_core = nchunk // NC
    kmax = (per_core + NS - 1) // NS
    rpt_a = -(-n_nodes // NS) + 7 & ~7
    rpt_lo = rpt_a * (NS - 1)
    rpt_b = n_nodes - rpt_lo
    mesh = plsc.VectorSubcoreMesh(core_axis_name="c", subcore_axis_name="s")

    out_type = [jax.ShapeDtypeStruct((NC, n_nodes, H), jnp.float32)]
    scratch = [
        pltpu.VMEM((CH,), jnp.int32),
        pltpu.VMEM((CH, H), jnp.float32),
        pltpu.VMEM_SHARED((n_nodes, H), jnp.float32),
    ]
    if has_coord:
        out_type += [jax.ShapeDtypeStruct((n_nodes,), jnp.float32)] * 10
        scratch += [
            pltpu.VMEM((CH,), jnp.int32),
            pltpu.VMEM((CH,), jnp.float32),
            pltpu.VMEM((CH,), jnp.float32),
            pltpu.VMEM((CH,), jnp.float32),
            pltpu.VMEM((CH,), jnp.float32),
            pltpu.VMEM((CH,), jnp.float32),
            pltpu.VMEM((CH,), jnp.float32),
            pltpu.VMEM((CH,), jnp.float32),
            pltpu.VMEM((CH,), jnp.float32),
            pltpu.VMEM_SHARED((n_nodes,), jnp.float32),
            pltpu.VMEM_SHARED((n_nodes,), jnp.float32),
            pltpu.VMEM_SHARED((n_nodes,), jnp.float32),
            pltpu.VMEM_SHARED((n_nodes,), jnp.float32),
            pltpu.VMEM_SHARED((n_nodes,), jnp.float32),
            pltpu.SemaphoreType.DMA,
        ]

    @functools.partial(pl.kernel, out_type=out_type, mesh=mesh,
                       scratch_types=scratch)
    def k(*refs):
        if has_coord:
            (m_h, row_h, z_h, qpk_h, col_h, x0_h, x1_h, x2_h, z1_h,
             accm_out, g0, g1, g2, g3, g4, g5, g6, g7, g8, g9,
             idx, mbuf, accm,
             idxc, qv, xc0, xc1, xc2, qx0, qx1, qx2, ones,
             aq0, aq1, aq2, aq3, adeg, sem) = refs
        else:
            m_h, row_h, z_h, accm_out, idx, mbuf, accm = refs
        c = lax.axis_index("c")
        s = lax.axis_index("s")
        off = pl.multiple_of(s * rpt_a, 8)

        @pl.when(s < NS - 1)
        def _():
            pltpu.sync_copy(z_h.at[pl.ds(off, rpt_a)],
                            accm.at[pl.ds(off, rpt_a)])

        @pl.when(s == NS - 1)
        def _():
            pltpu.sync_copy(z_h.at[pl.ds(rpt_lo, rpt_b)],
                            accm.at[pl.ds(rpt_lo, rpt_b)])
        if has_coord:
            for g in range(CH // 16):
                ones[pl.ds(g * 16, 16)] = jnp.full((16,), 1.0, jnp.float32)

            @pl.when(s == 0)
            def _():
                for a in (aq0, aq1, aq2, aq3, adeg):
                    pltpu.sync_copy(z1_h, a)
        plsc.subcore_barrier()

        def body(kk, carry):
            @pl.when(s + NS * kk < per_core)
            def _():
                t = c * per_core + s + NS * kk
                base = pl.multiple_of(t * CH, CH)
                pltpu.sync_copy(row_h.at[pl.ds(base, CH)], idx)
                pltpu.sync_copy(m_h.at[pl.ds(base, CH)], mbuf)
                pltpu.sync_copy(mbuf, accm.at[idx], add=True)
                if has_coord:
                    pltpu.sync_copy(col_h.at[pl.ds(base, CH)], idxc)
                    pltpu.sync_copy(qpk_h.at[t], qv)
                    c0 = pltpu.async_copy(x0_h.at[idxc], xc0, sem)
                    c0.wait()
                    c1 = pltpu.async_copy(x1_h.at[idxc], xc1, sem)
                    c1.wait()
                    c2 = pltpu.async_copy(x2_h.at[idxc], xc2, sem)
                    c2.wait()
                    for g in range(CH // 16):
                        d = pl.ds(g * 16, 16)
                        q = qv[d]
                        qx0[d] = q * xc0[d]
                        qx1[d] = q * xc1[d]
                        qx2[d] = q * xc2[d]
                    pltpu.sync_copy(qx0, aq0.at[idx], add=True)
                    pltpu.sync_copy(qx1, aq1.at[idx], add=True)
                    pltpu.sync_copy(qx2, aq2.at[idx], add=True)
                    pltpu.sync_copy(qv, aq3.at[idx], add=True)
                    pltpu.sync_copy(ones, adeg.at[idx], add=True)
            return carry

        lax.fori_loop(0, kmax, body, 0)
        plsc.subcore_barrier()

        @pl.when(s < NS - 1)
        def _():
            pltpu.sync_copy(accm.at[pl.ds(off, rpt_a)],
                            accm_out.at[c, pl.ds(off, rpt_a)])

        @pl.when(s == NS - 1)
        def _():
            pltpu.sync_copy(accm.at[pl.ds(rpt_lo, rpt_b)],
                            accm_out.at[c, pl.ds(rpt_lo, rpt_b)])
        if has_coord:
            @pl.when((s == 1) & (c == 0))
            def _():
                for dst, a in zip((g0, g1, g2, g3, g4),
                                  (aq0, aq1, aq2, aq3, adeg)):
                    pltpu.sync_copy(a, dst)

            @pl.when((s == 1) & (c == 1))
            def _():
                for dst, a in zip((g5, g6, g7, g8, g9),
                                  (aq0, aq1, aq2, aq3, adeg)):
                    pltpu.sync_copy(a, dst)

    if has_coord:
        res = k(medge, row, zeros_m, qpk, col, x0, x1, x2, zeros_1)
        return res[0], jnp.stack(res[1:])
    return k(medge, row, zeros_m)


def _full(shape):
    nd = len(shape)
    return pl.BlockSpec(shape, lambda i: (0,) * nd)


def _enc_node_prep(h_in, ne_w, ne_b, ne_g, ne_beta, w1a, w1b, eb1):
    n = h_in.shape[0]
    nf = h_in.shape[1]

    def body(h_ref, w_ref, b_ref, g_ref, beta_ref, wa_ref, wb_ref, eb_ref,
             h0_ref, a_ref, b2_ref):
        h0 = _ln(_silu(jnp.dot(h_ref[...], w_ref[...],
                               preferred_element_type=jnp.float32) + b_ref[...]),
                 g_ref[...], beta_ref[...])
        h0_ref[...] = h0
        a_ref[...] = jnp.dot(h0, wa_ref[...],
                             preferred_element_type=jnp.float32) + eb_ref[...]
        b2_ref[...] = jnp.dot(h0, wb_ref[...],
                              preferred_element_type=jnp.float32)

    return pl.pallas_call(
        body,
        grid=(n // BN,),
        in_specs=[pl.BlockSpec((BN, nf), lambda i: (i, 0)),
                  _full((nf, H)), _full((1, H)), _full((1, H)), _full((1, H)),
                  _full((H, H)), _full((H, H)), _full((1, H))],
        out_specs=[pl.BlockSpec((BN, H), lambda i: (i, 0))] * 3,
        out_shape=[jax.ShapeDtypeStruct((n, H), jnp.float32)] * 3,
    )(h_in, ne_w, ne_b, ne_g, ne_beta, w1a, w1b, eb1)


def _enc_edge(edge_attr, ee_w, ee_b):
    e = edge_attr.shape[0]
    ef = edge_attr.shape[1]

    def body(ea_ref, w_ref, b_ref, out_ref):
        out_ref[...] = _silu(jnp.dot(ea_ref[...], w_ref[...],
                                     preferred_element_type=jnp.float32)
                             + b_ref[...])

    return pl.pallas_call(
        body,
        grid=(e // BE,),
        in_specs=[pl.BlockSpec((BE, ef), lambda i: (i, 0)),
                  _full((ef, H)), _full((1, H))],
        out_specs=pl.BlockSpec((BE, H), lambda i: (i, 0)),
        out_shape=jax.ShapeDtypeStruct((e, H), jnp.float32),
    )(edge_attr, ee_w, ee_b)


def _tc_edge(sg, dsqp, ea, w1d, w1e, ew2, eb2, cw1, cb1, cw2, has_coord):
    e = sg.shape[0]
    nrow = BE // CH

    def body(sg_ref, dsq_ref, ea_ref, w1d_ref, w1e_ref, ew2_ref, eb2_ref,
             cw1_ref, cb1_ref, cw2_ref, *outs):
        eye = jnp.eye(CH, dtype=jnp.float32)
        packed = dsq_ref[...][0]
        cols = lax.dot_general(eye, packed, (((1,), (1,)), ((), ())),
                               preferred_element_type=jnp.float32)
        dsq = jnp.concatenate([cols[:, r:r + 1] for r in range(nrow)],
                              axis=0)
        pre = (sg_ref[...] + dsq * w1d_ref[...]
               + jnp.dot(ea_ref[...], w1e_ref[...],
                         preferred_element_type=jnp.float32))
        m1 = _silu(pre)
        m = _silu(jnp.dot(m1, ew2_ref[...],
                          preferred_element_type=jnp.float32) + eb2_ref[...])
        outs[0][...] = m
        if has_coord:
            t = _silu(jnp.dot(m, cw1_ref[...],
                              preferred_element_type=jnp.float32) + cb1_ref[...])
            cw = jnp.tanh(jnp.dot(t, cw2_ref[...],
                                  preferred_element_type=jnp.float32))
            q = cw * lax.rsqrt(dsq + 1e-8)
            qcols = jnp.concatenate(
                [q[r * CH:(r + 1) * CH] for r in range(nrow)], axis=1)
            qpacked = lax.dot_general(qcols, eye, (((0,), (0,)), ((), ())),
                                      preferred_element_type=jnp.float32)
            outs[1][...] = qpacked[None]

    out_specs = [pl.BlockSpec((BE, H), lambda i: (i, 0))]
    out_shape = [jax.ShapeDtypeStruct((e, H), jnp.float32)]
    if has_coord:
        out_specs.append(pl.BlockSpec((1, nrow, CH), lambda i: (i, 0, 0)))
        out_shape.append(jax.ShapeDtypeStruct((e // BE, nrow, CH), jnp.float32))
    res = pl.pallas_call(
        body,
        grid=(e // BE,),
        in_specs=[pl.BlockSpec((BE, H), lambda i: (i, 0)),
                  pl.BlockSpec((1, nrow, CH), lambda i: (i, 0, 0)),
                  pl.BlockSpec((BE, H), lambda i: (i, 0)),
                  _full((1, H)), _full((H, H)), _full((H, H)), _full((1, H)),
                  _full((H, H)), _full((1, H)), _full((H, 1))],
        out_specs=out_specs,
        out_shape=out_shape,
    )(sg, jnp.reshape(dsqp, (e // BE, nrow, CH)), ea,
      w1d, w1e, ew2, eb2, cw1, cb1, cw2)
    if has_coord:
        return res[0], jnp.reshape(res[1], (e // CH, CH))
    return res[0], None


def _tc_xupdate(x3, accg):
    n = x3.shape[1]

    def body(x_ref, g_ref, xn_ref):
        ag = g_ref[...]
        xv = x_ref[...]
        s1 = ag[3:4] + ag[8:9]
        deg = jnp.maximum(ag[4:5] + ag[9:10], 1.0)
        sxc = ag[0:3] + ag[5:8]
        xn_ref[...] = xv + (xv * s1 - sxc) / deg

    return pl.pallas_call(
        body,
        out_shape=jax.ShapeDtypeStruct((3, n), jnp.float32),
    )(x3, accg)


def _tc_node(h, am0, am1, nw1h, nw1m, nb1, nw2, nb2, ln_g, ln_b,
             w1a, w1b, eb1):
    n = h.shape[0]

    def body(h_ref, a0_ref, a1_ref, w1h_ref, w1m_ref, b1_ref,
             w2_ref, b2_ref, lg_ref, lb_ref, wa_ref, wb_ref, eb_ref,
             hn_ref, ta_ref, tb_ref):
        hv = h_ref[...]
        m_i = a0_ref[...] + a1_ref[...]
        pre = (jnp.dot(hv, w1h_ref[...], preferred_element_type=jnp.float32)
               + jnp.dot(m_i, w1m_ref[...], preferred_element_type=jnp.float32)
               + b1_ref[...])
        hu = jnp.dot(_silu(pre), w2_ref[...],
                     preferred_element_type=jnp.float32) + b2_ref[...]
        hn = _ln(hv + hu, lg_ref[...], lb_ref[...])
        hn_ref[...] = hn
        ta_ref[...] = jnp.dot(hn, wa_ref[...],
                              preferred_element_type=jnp.float32) + eb_ref[...]
        tb_ref[...] = jnp.dot(hn, wb_ref[...],
                              preferred_element_type=jnp.float32)

    return pl.pallas_call(
        body,
        grid=(n // BN,),
        in_specs=[pl.BlockSpec((BN, H), lambda i: (i, 0)),
                  pl.BlockSpec((BN, H), lambda i: (i, 0)),
                  pl.BlockSpec((BN, H), lambda i: (i, 0)),
                  _full((H, H)), _full((H, H)), _full((1, H)),
                  _full((H, H)), _full((1, H)), _full((1, H)), _full((1, H)),
                  _full((H, H)), _full((H, H)), _full((1, H))],
        out_specs=[pl.BlockSpec((BN, H), lambda i: (i, 0)),
                   pl.BlockSpec((BN, H), lambda i: (i, 0)),
                   pl.BlockSpec((BN, H), lambda i: (i, 0))],
        out_shape=[jax.ShapeDtypeStruct((n, H), jnp.float32),
                   jax.ShapeDtypeStruct((n, H), jnp.float32),
                   jax.ShapeDtypeStruct((n, H), jnp.float32)],
    )(h, am0, am1, nw1h, nw1m, nb1, nw2, nb2, ln_g, ln_b, w1a, w1b, eb1)


def _tc_node_last(h, am0, am1, nw1h, nw1m, nb1, nw2, nb2, ln_g, ln_b,
                  pw1, pb1, pw2, pb2):
    n = h.shape[0]

    def body(h_ref, a0_ref, a1_ref, w1h_ref, w1m_ref, b1_ref, w2_ref, b2_ref,
             lg_ref, lb_ref, pw1_ref, pb1_ref, pw2_ref, pb2_ref,
             hn_ref, s_ref):
        hv = h_ref[...]
        m_i = a0_ref[...] + a1_ref[...]
        pre = (jnp.dot(hv, w1h_ref[...], preferred_element_type=jnp.float32)
               + jnp.dot(m_i, w1m_ref[...], preferred_element_type=jnp.float32)
               + b1_ref[...])
        hu = jnp.dot(_silu(pre), w2_ref[...],
                     preferred_element_type=jnp.float32) + b2_ref[...]
        hn = _ln(hv + hu, lg_ref[...], lb_ref[...])
        hn_ref[...] = hn
        t = jnp.tanh(jnp.dot(hn, pw1_ref[...],
                             preferred_element_type=jnp.float32) + pb1_ref[...])
        s_ref[...] = jnp.dot(t, pw2_ref[...],
                             preferred_element_type=jnp.float32) + pb2_ref[...]

    return pl.pallas_call(
        body,
        grid=(n // BN,),
        in_specs=[pl.BlockSpec((BN, H), lambda i: (i, 0)),
                  pl.BlockSpec((BN, H), lambda i: (i, 0)),
                  pl.BlockSpec((BN, H), lambda i: (i, 0)),
                  _full((H, H)), _full((H, H)), _full((1, H)),
                  _full((H, H)), _full((1, H)), _full((1, H)), _full((1, H)),
                  _full((H, H)), _full((1, H)), _full((H, 1)), _full((1, 1))],
        out_specs=[pl.BlockSpec((BN, H), lambda i: (i, 0)),
                   pl.BlockSpec((BN, 1), lambda i: (i, 0))],
        out_shape=[jax.ShapeDtypeStruct((n, H), jnp.float32),
                   jax.ShapeDtypeStruct((n, 1), jnp.float32)],
    )(h, am0, am1, nw1h, nw1m, nb1, nw2, nb2, ln_g, ln_b, pw1, pb1, pw2, pb2)


def _tc_pool(h4, sarr, batch2d, nb, cw1, cb1, cw2, cb2, cw3, cb3):
    n = h4.shape[0]

    def body(h_ref, s_ref, b_ref, cw1_ref, cb1_ref, cw2_ref, cb2_ref,
             cw3_ref, cb3_ref, out_ref):
        bv = b_ref[...]
        onehot = (bv == lax.broadcasted_iota(jnp.int32, (1, nb), 1))
        sv = s_ref[...]
        s_b = jnp.broadcast_to(sv, (n, nb))
        smax = jnp.max(jnp.where(onehot, s_b, -1e30), axis=0, keepdims=True)
        mm = jnp.where(onehot, jnp.exp(s_b - jnp.broadcast_to(smax, (n, nb))),
                       0.0)
        ssum = lax.dot_general(mm, jnp.ones((n, 1), jnp.float32),
                               (((0,), (0,)), ((), ())),
                               preferred_element_type=jnp.float32)
        gnum = lax.dot_general(mm, h_ref[...], (((0,), (0,)), ((), ())),
                               preferred_element_type=jnp.float32)
        g = gnum / (ssum + 1e-16)
        inv = 1.0 / jnp.sqrt(jnp.float32(1.0 + 1e-5))
        z = _silu(jnp.dot(g, cw1_ref[...],
                          preferred_element_type=jnp.float32) + cb1_ref[...]) * inv
        z = _silu(jnp.dot(z, cw2_ref[...],
                          preferred_element_type=jnp.float32) + cb2_ref[...]) * inv
        out_ref[...] = jnp.dot(z, cw3_ref[...],
                               preferred_element_type=jnp.float32) + cb3_ref[...]

    return pl.pallas_call(
        body,
        out_shape=jax.ShapeDtypeStruct((nb, 1), jnp.float32),
    )(h4, sarr, batch2d, cw1, cb1, cw2, cb2, cw3, cb3)


def kernel(h, x, edge_index, edge_attr, batch, params):
    n = h.shape[0]
    row = edge_index[0]
    col = edge_index[1]
    x0, x1, x2 = x[:, 0], x[:, 1], x[:, 2]
    p = params
    layers = p["layers"]

    def r2(v):
        return v.reshape(1, -1)

    def split_ew1(lp):
        ew1 = lp["ew1"]
        return ew1[:H], ew1[H:2 * H], ew1[2 * H:2 * H + 1], ew1[2 * H + 1:]

    w1a0, w1b0, _, _ = split_ew1(layers[0])
    hcur, tab_a, tab_b = _enc_node_prep(
        h, p["ne_w"], r2(p["ne_b"]), r2(p["ne_g"]), r2(p["ne_beta"]),
        w1a0, w1b0, r2(layers[0]["eb1"]))
    ea = _enc_edge(edge_attr, p["ee_w"], r2(p["ee_b"]))
    zeros_m = jnp.zeros((n, H), jnp.float32)
    zeros_1 = jnp.zeros((n,), jnp.float32)
    sarr = None
    for i, lp in enumerate(layers):
        _, _, w1d, w1e = split_ew1(lp)
        has_coord = i < len(layers) - 1
        sg, dsqp = _sc_gather(tab_a, tab_b, row, col, x0, x1, x2)
        if has_coord:
            cw1, cb1, cw2 = lp["cw1"], r2(lp["cb1"]), lp["cw2"]
        else:
            cw1 = jnp.zeros((H, H), jnp.float32)
            cb1 = jnp.zeros((1, H), jnp.float32)
            cw2 = jnp.zeros((H, 1), jnp.float32)
        medge, qpk = _tc_edge(sg, dsqp, ea, w1d, w1e, lp["ew2"], r2(lp["eb2"]),
                              cw1, cb1, cw2, has_coord)
        nw1 = lp["nw1"]
        if has_coord:
            acc_m, acc_g = _sc_scatter(medge, row, zeros_m, qpk, col,
                                       x0, x1, x2, zeros_1)
            nlp = layers[i + 1]
            w1a, w1b, _, _ = split_ew1(nlp)
            xn3 = _tc_xupdate(jnp.stack([x0, x1, x2]), acc_g)
            x0, x1, x2 = xn3[0], xn3[1], xn3[2]
            hcur, tab_a, tab_b = _tc_node(
                hcur, acc_m[0], acc_m[1],
                nw1[:H], nw1[H:], r2(lp["nb1"]), lp["nw2"], r2(lp["nb2"]),
                r2(lp["ln_g"]), r2(lp["ln_b"]),
                w1a, w1b, r2(nlp["eb1"]))
        else:
            (acc_m,) = _sc_scatter(medge, row, zeros_m)
            hcur, sarr = _tc_node_last(
                hcur, acc_m[0], acc_m[1],
                nw1[:H], nw1[H:], r2(lp["nb1"]), lp["nw2"], r2(lp["nb2"]),
                r2(lp["ln_g"]), r2(lp["ln_b"]),
                p["pw1"], r2(p["pb1"]), p["pw2"], r2(p["pb2"]))
    nb = 64
    return _tc_pool(hcur, sarr, batch.reshape(n, 1), nb,
                    p["cw1"], r2(p["cb1"]), p["cw2"], r2(p["cb2"]),
                    p["cw3"], r2(p["cb3"]))


# speedup vs baseline: 4.3925x; 1.3805x over previous
import functools

import jax
import jax.numpy as jnp
from jax import lax
from jax.experimental import pallas as pl
from jax.experimental.pallas import tpu as pltpu
from jax.experimental.pallas import tpu_sc as plsc

NC = 2
NS = 16
CH = 128
H = 128
BE = 1280
BN = 2000


def _silu(t):
    return t * jax.nn.sigmoid(t)


def _ln(t, g, b):
    mu = jnp.mean(t, -1, keepdims=True)
    var = jnp.mean((t - mu) ** 2, -1, keepdims=True)
    return (t - mu) * lax.rsqrt(var + 1e-5) * g + b


def _sc_gather(tab_a, tab_b, row, col, x0, x1, x2):
    n_edges = row.shape[0]
    nchunk = n_edges // CH
    per_core = nchunk // NC
    per_tile = per_core // NS
    ngrp = per_tile // 2
    n_extra = per_core - per_tile * NS
    mesh = plsc.VectorSubcoreMesh(core_axis_name="c", subcore_axis_name="s")

    @functools.partial(
        pl.kernel,
        out_type=[jax.ShapeDtypeStruct((n_edges, H), jnp.float32),
                  jax.ShapeDtypeStruct((nchunk, CH), jnp.float32)],
        mesh=mesh,
        scratch_types=[
            pltpu.VMEM((2, CH), jnp.int32),
            pltpu.VMEM((2, CH), jnp.int32),
            pltpu.VMEM((2, CH, H), jnp.float32),
            pltpu.VMEM((2, CH), jnp.float32),
            pltpu.VMEM((2, CH), jnp.float32),
            pltpu.VMEM((2, CH), jnp.float32),
            pltpu.VMEM((2, CH), jnp.float32),
            pltpu.VMEM((2, CH), jnp.float32),
            pltpu.VMEM((2, CH), jnp.float32),
            pltpu.VMEM((2, CH), jnp.float32),
        ] + [pltpu.SemaphoreType.DMA] * 10,
    )
    def k(ta, tb, row_h, col_h, x0_h, x1_h, x2_h, sg_h, dsq_h,
          idxr, idxc, buf, xr0, xr1, xr2, xc0, xc1, xc2, dsqv,
          sl0, sl1, sa0, sa1, sb0, sb1, sx0, sx1, sw0, sw1):
        c = lax.axis_index("c")
        s = lax.axis_index("s")
        start = c * per_core + s * per_tile
        sl = (sl0, sl1)
        sa = (sa0, sa1)
        sb = (sb0, sb1)
        sx = (sx0, sx1)
        sw = (sw0, sw1)

        def base_of(t):
            return pl.multiple_of(t * CH, CH)

        def grp(g, carry):
            bases = [base_of(start + g * 2 + b) for b in (0, 1)]
            @pl.when(g > 0)
            def _():
                for b in (0, 1):
                    pltpu.make_async_copy(buf.at[b], sg_h.at[pl.ds(0, CH)],
                                          sw[b]).wait()
                    pltpu.make_async_copy(dsqv.at[b], dsq_h.at[0],
                                          sw[b]).wait()
            ls = []
            for b in (0, 1):
                ls.append(pltpu.async_copy(row_h.at[pl.ds(bases[b], CH)],
                                           idxr.at[b], sl[b]))
                ls.append(pltpu.async_copy(col_h.at[pl.ds(bases[b], CH)],
                                           idxc.at[b], sl[b]))
            cas, xs = [], []
            for b in (0, 1):
                ls[2 * b].wait()
                ls[2 * b + 1].wait()
                cas.append(pltpu.async_copy(ta.at[idxr.at[b]], buf.at[b],
                                            sa[b]))
                xs.append((pltpu.async_copy(x0_h.at[idxr.at[b]], xr0.at[b], sx[b]),
                           pltpu.async_copy(x1_h.at[idxr.at[b]], xr1.at[b], sx[b]),
                           pltpu.async_copy(x2_h.at[idxr.at[b]], xr2.at[b], sx[b]),
                           pltpu.async_copy(x0_h.at[idxc.at[b]], xc0.at[b], sx[b]),
                           pltpu.async_copy(x1_h.at[idxc.at[b]], xc1.at[b], sx[b]),
                           pltpu.async_copy(x2_h.at[idxc.at[b]], xc2.at[b], sx[b])))
            cbs = []
            for b in (0, 1):
                cas[b].wait()
                cbs.append(pltpu.async_copy(tb.at[idxc.at[b]], buf.at[b],
                                            sb[b], add=True))
            for b in (0, 1):
                for xcp in xs[b]:
                    xcp.wait()
                for g16 in range(CH // 16):
                    d = pl.ds(g16 * 16, 16)
                    a = xr0[b, d] - xc0[b, d]
                    bb = xr1[b, d] - xc1[b, d]
                    cc = xr2[b, d] - xc2[b, d]
                    dsqv[b, d] = a * a + bb * bb + cc * cc
            for b in (0, 1):
                cbs[b].wait()
                pltpu.async_copy(buf.at[b], sg_h.at[pl.ds(bases[b], CH)], sw[b])
                pltpu.async_copy(dsqv.at[b], dsq_h.at[start + g * 2 + b], sw[b])
            return carry

        lax.fori_loop(0, ngrp, grp, 0)
        for b in (0, 1):
            pltpu.make_async_copy(buf.at[b], sg_h.at[pl.ds(0, CH)], sw[b]).wait()
            pltpu.make_async_copy(dsqv.at[b], dsq_h.at[0], sw[b]).wait()

        @pl.when(s < n_extra)
        def _():
            t = c * per_core + NS * per_tile + s
            base = base_of(t)
            pltpu.sync_copy(row_h.at[pl.ds(base, CH)], idxr.at[0])
            pltpu.sync_copy(col_h.at[pl.ds(base, CH)], idxc.at[0])
            ca = pltpu.async_copy(ta.at[idxr.at[0]], buf.at[0], sa0)
            g0 = pltpu.async_copy(x0_h.at[idxr.at[0]], xr0.at[0], sx0)
            g1 = pltpu.async_copy(x1_h.at[idxr.at[0]], xr1.at[0], sx0)
            g2 = pltpu.async_copy(x2_h.at[idxr.at[0]], xr2.at[0], sx0)
            g3 = pltpu.async_copy(x0_h.at[idxc.at[0]], xc0.at[0], sx0)
            g4 = pltpu.async_copy(x1_h.at[idxc.at[0]], xc1.at[0], sx0)
            g5 = pltpu.async_copy(x2_h.at[idxc.at[0]], xc2.at[0], sx0)
            ca.wait()
            cb = pltpu.async_copy(tb.at[idxc.at[0]], buf.at[0], sb0, add=True)
            g0.wait(); g1.wait(); g2.wait()
            g3.wait(); g4.wait(); g5.wait()
            for g16 in range(CH // 16):
                d = pl.ds(g16 * 16, 16)
                a = xr0[0, d] - xc0[0, d]
                bb = xr1[0, d] - xc1[0, d]
                cc = xr2[0, d] - xc2[0, d]
                dsqv[0, d] = a * a + bb * bb + cc * cc
            cb.wait()
            pltpu.sync_copy(buf.at[0], sg_h.at[pl.ds(base, CH)])
            pltpu.sync_copy(dsqv.at[0], dsq_h.at[t])

    return k(tab_a, tab_b, row, col, x0, x1, x2)


def _sc_scatter(medge, row, zeros_m, qpk=None, col=None, x0=None, x1=None,
                x2=None, zeros_1=None):
    has_coord = qpk is not None
    n_edges = row.shape[0]
    n_nodes = zeros_m.shape[0]
    nchunk = n_edges // CH
    per_core = nchunk // NC
    kmax = (per_core + NS - 1) // NS
    rpt_a = -(-n_nodes // NS) + 7 & ~7
    rpt_lo = rpt_a * (NS - 1)
    rpt_b = n_nodes - rpt_lo
    mesh = plsc.VectorSubcoreMesh(core_axis_name="c", subcore_axis_name="s")

    per_tile = per_core // NS
    ngrp = per_tile // 2
    n_extra = per_core - per_tile * NS
    out_type = [jax.ShapeDtypeStruct((NC, n_nodes, H), jnp.float32)]
    scratch = [
        pltpu.VMEM((2, CH), jnp.int32),
        pltpu.VMEM((2, CH, H), jnp.float32),
        pltpu.VMEM_SHARED((n_nodes, H), jnp.float32),
    ]
    if has_coord:
        out_type += [jax.ShapeDtypeStruct((n_nodes,), jnp.float32)] * 10
        scratch += [
            pltpu.VMEM((2, CH), jnp.int32),
            pltpu.VMEM((2, CH), jnp.float32),
            pltpu.VMEM((2, CH), jnp.float32),
            pltpu.VMEM((2, CH), jnp.float32),
            pltpu.VMEM((2, CH), jnp.float32),
            pltpu.VMEM((2, CH), jnp.float32),
            pltpu.VMEM((2, CH), jnp.float32),
            pltpu.VMEM((2, CH), jnp.float32),
            pltpu.VMEM((CH,), jnp.float32),
            pltpu.VMEM_SHARED((n_nodes,), jnp.float32),
            pltpu.VMEM_SHARED((n_nodes,), jnp.float32),
            pltpu.VMEM_SHARED((n_nodes,), jnp.float32),
            pltpu.VMEM_SHARED((n_nodes,), jnp.float32),
            pltpu.VMEM_SHARED((n_nodes,), jnp.float32),
        ]
    scratch += [pltpu.SemaphoreType.DMA] * 6

    @functools.partial(pl.kernel, out_type=out_type, mesh=mesh,
                       scratch_types=scratch)
    def k(*refs):
        if has_coord:
            (m_h, row_h, z_h, qpk_h, col_h, x0_h, x1_h, x2_h, z1_h,
             accm_out, g0, g1, g2, g3, g4, g5, g6, g7, g8, g9,
             idx, mbuf, accm,
             idxc, qv, xc0, xc1, xc2, qx0, qx1, qx2, ones,
             aq0, aq1, aq2, aq3, adeg,
             sml0, sml1, sxg0, sxg1, ssa0, ssa1) = refs
        else:
            (m_h, row_h, z_h, accm_out, idx, mbuf, accm,
             sml0, sml1, sxg0, sxg1, ssa0, ssa1) = refs
        c = lax.axis_index("c")
        s = lax.axis_index("s")
        start = c * per_core + s * per_tile
        sml = (sml0, sml1)
        sxg = (sxg0, sxg1)
        ssa = (ssa0, ssa1)
        off = pl.multiple_of(s * rpt_a, 8)

        @pl.when(s < NS - 1)
        def _():
            pltpu.sync_copy(z_h.at[pl.ds(off, rpt_a)],
                            accm.at[pl.ds(off, rpt_a)])

        @pl.when(s == NS - 1)
        def _():
            pltpu.sync_copy(z_h.at[pl.ds(rpt_lo, rpt_b)],
                            accm.at[pl.ds(rpt_lo, rpt_b)])
        if has_coord:
            for g in range(CH // 16):
                ones[pl.ds(g * 16, 16)] = jnp.full((16,), 1.0, jnp.float32)

            @pl.when(s == 0)
            def _():
                for a in (aq0, aq1, aq2, aq3, adeg):
                    pltpu.sync_copy(z1_h, a)
        plsc.subcore_barrier()

        def drain(b):
            pltpu.make_async_copy(mbuf.at[b], accm.at[pl.ds(0, CH)],
                                  ssa[b]).wait()
            if has_coord:
                for src, dst in ((qx0, aq0), (qx1, aq1), (qx2, aq2),
                                 (qv, aq3), (qv, adeg)):
                    pltpu.make_async_copy(src.at[b], dst.at[pl.ds(0, CH)],
                                          ssa[b]).wait()

        def do_chunk_loads(t, b):
            base = pl.multiple_of(t * CH, CH)
            lds = [pltpu.async_copy(row_h.at[pl.ds(base, CH)], idx.at[b],
                                    sml[b]),
                   pltpu.async_copy(m_h.at[pl.ds(base, CH)], mbuf.at[b],
                                    sml[b])]
            if has_coord:
                lds.append(pltpu.async_copy(col_h.at[pl.ds(base, CH)],
                                            idxc.at[b], sml[b]))
                lds.append(pltpu.async_copy(qpk_h.at[t], qv.at[b], sml[b]))
            return lds

        def grp(g, carry):
            @pl.when(g > 0)
            def _():
                for b in (0, 1):
                    drain(b)
            lds = [do_chunk_loads(start + g * 2 + b, b) for b in (0, 1)]
            xg = []
            for b in (0, 1):
                for ld in lds[b]:
                    ld.wait()
                pltpu.async_copy(mbuf.at[b], accm.at[idx.at[b]], ssa[b],
                                 add=True)
                if has_coord:
                    xg.append((
                        pltpu.async_copy(x0_h.at[idxc.at[b]], xc0.at[b], sxg[b]),
                        pltpu.async_copy(x1_h.at[idxc.at[b]], xc1.at[b], sxg[b]),
                        pltpu.async_copy(x2_h.at[idxc.at[b]], xc2.at[b], sxg[b])))
            if has_coord:
                for b in (0, 1):
                    for xcp in xg[b]:
                        xcp.wait()
                    for g16 in range(CH // 16):
                        d = pl.ds(g16 * 16, 16)
                        q = qv[b, d]
                        qx0[b, d] = q * xc0[b, d]
                        qx1[b, d] = q * xc1[b, d]
                        qx2[b, d] = q * xc2[b, d]
                    pltpu.async_copy(qx0.at[b], aq0.at[idx.at[b]], ssa[b],
                                     add=True)
                    pltpu.async_copy(qx1.at[b], aq1.at[idx.at[b]], ssa[b],
                                     add=True)
                    pltpu.async_copy(qx2.at[b], aq2.at[idx.at[b]], ssa[b],
                                     add=True)
                    pltpu.async_copy(qv.at[b], aq3.at[idx.at[b]], ssa[b],
                                     add=True)
                    pltpu.async_copy(ones, adeg.at[idx.at[b]], ssa[b],
                                     add=True)
            return carry

        lax.fori_loop(0, ngrp, grp, 0)
        for b in (0, 1):
            drain(b)

        @pl.when(s < n_extra)
        def _():
            t = c * per_core + NS * per_tile + s
            base = pl.multiple_of(t * CH, CH)
            pltpu.sync_copy(row_h.at[pl.ds(base, CH)], idx.at[0])
            pltpu.sync_copy(m_h.at[pl.ds(base, CH)], mbuf.at[0])
            pltpu.sync_copy(mbuf.at[0], accm.at[idx.at[0]], add=True)
            if has_coord:
                pltpu.sync_copy(col_h.at[pl.ds(base, CH)], idxc.at[0])
                pltpu.sync_copy(qpk_h.at[t], qv.at[0])
                c0 = pltpu.async_copy(x0_h.at[idxc.at[0]], xc0.at[0], sxg0)
                c1 = pltpu.async_copy(x1_h.at[idxc.at[0]], xc1.at[0], sxg0)
                c2 = pltpu.async_copy(x2_h.at[idxc.at[0]], xc2.at[0], sxg0)
                c0.wait(); c1.wait(); c2.wait()
                for g16 in range(CH // 16):
                    d = pl.ds(g16 * 16, 16)
                    q = qv[0, d]
                    qx0[0, d] = q * xc0[0, d]
                    qx1[0, d] = q * xc1[0, d]
                    qx2[0, d] = q * xc2[0, d]
                pltpu.sync_copy(qx0.at[0], aq0.at[idx.at[0]], add=True)
                pltpu.sync_copy(qx1.at[0], aq1.at[idx.at[0]], add=True)
                pltpu.sync_copy(qx2.at[0], aq2.at[idx.at[0]], add=True)
                pltpu.sync_copy(qv.at[0], aq3.at[idx.at[0]], add=True)
                pltpu.sync_copy(ones, adeg.at[idx.at[0]], add=True)
        plsc.subcore_barrier()

        @pl.when(s < NS - 1)
        def _():
            pltpu.sync_copy(accm.at[pl.ds(off, rpt_a)],
                            accm_out.at[c, pl.ds(off, rpt_a)])

        @pl.when(s == NS - 1)
        def _():
            pltpu.sync_copy(accm.at[pl.ds(rpt_lo, rpt_b)],
                            accm_out.at[c, pl.ds(rpt_lo, rpt_b)])
        if has_coord:
            @pl.when((s == 1) & (c == 0))
            def _():
                for dst, a in zip((g0, g1, g2, g3, g4),
                                  (aq0, aq1, aq2, aq3, adeg)):
                    pltpu.sync_copy(a, dst)

            @pl.when((s == 1) & (c == 1))
            def _():
                for dst, a in zip((g5, g6, g7, g8, g9),
                                  (aq0, aq1, aq2, aq3, adeg)):
                    pltpu.sync_copy(a, dst)

    if has_coord:
        res = k(medge, row, zeros_m, qpk, col, x0, x1, x2, zeros_1)
        return res[0], jnp.stack(res[1:])
    return k(medge, row, zeros_m)


def _full(shape):
    nd = len(shape)
    return pl.BlockSpec(shape, lambda i: (0,) * nd)


def _enc_node_prep(h_in, ne_w, ne_b, ne_g, ne_beta, w1a, w1b, eb1):
    n = h_in.shape[0]
    nf = h_in.shape[1]

    def body(h_ref, w_ref, b_ref, g_ref, beta_ref, wa_ref, wb_ref, eb_ref,
             h0_ref, a_ref, b2_ref):
        h0 = _ln(_silu(jnp.dot(h_ref[...], w_ref[...],
                               preferred_element_type=jnp.float32) + b_ref[...]),
                 g_ref[...], beta_ref[...])
        h0_ref[...] = h0
        a_ref[...] = jnp.dot(h0, wa_ref[...],
                             preferred_element_type=jnp.float32) + eb_ref[...]
        b2_ref[...] = jnp.dot(h0, wb_ref[...],
                              preferred_element_type=jnp.float32)

    return pl.pallas_call(
        body,
        grid=(n // BN,),
        in_specs=[pl.BlockSpec((BN, nf), lambda i: (i, 0)),
                  _full((nf, H)), _full((1, H)), _full((1, H)), _full((1, H)),
                  _full((H, H)), _full((H, H)), _full((1, H))],
        out_specs=[pl.BlockSpec((BN, H), lambda i: (i, 0))] * 3,
        out_shape=[jax.ShapeDtypeStruct((n, H), jnp.float32)] * 3,
    )(h_in, ne_w, ne_b, ne_g, ne_beta, w1a, w1b, eb1)


def _enc_edge(edge_attr, ee_w, ee_b):
    e = edge_attr.shape[0]
    ef = edge_attr.shape[1]

    def body(ea_ref, w_ref, b_ref, out_ref):
        out_ref[...] = _silu(jnp.dot(ea_ref[...], w_ref[...],
                                     preferred_element_type=jnp.float32)
                             + b_ref[...])

    return pl.pallas_call(
        body,
        grid=(e // BE,),
        in_specs=[pl.BlockSpec((BE, ef), lambda i: (i, 0)),
                  _full((ef, H)), _full((1, H))],
        out_specs=pl.BlockSpec((BE, H), lambda i: (i, 0)),
        out_shape=jax.ShapeDtypeStruct((e, H), jnp.float32),
    )(edge_attr, ee_w, ee_b)


def _tc_edge(sg, dsqp, ea, w1d, w1e, ew2, eb2, cw1, cb1, cw2, has_coord):
    e = sg.shape[0]
    nrow = BE // CH

    def body(sg_ref, dsq_ref, ea_ref, w1d_ref, w1e_ref, ew2_ref, eb2_ref,
             cw1_ref, cb1_ref, cw2_ref, *outs):
        eye = jnp.eye(CH, dtype=jnp.float32)
        packed = dsq_ref[...][0]
        cols = lax.dot_general(eye, packed, (((1,), (1,)), ((), ())),
                               preferred_element_type=jnp.float32)
        dsq = jnp.concatenate([cols[:, r:r + 1] for r in range(nrow)],
                              axis=0)
        pre = (sg_ref[...] + dsq * w1d_ref[...]
               + jnp.dot(ea_ref[...], w1e_ref[...],
                         preferred_element_type=jnp.float32))
        m1 = _silu(pre)
        m = _silu(jnp.dot(m1, ew2_ref[...],
                          preferred_element_type=jnp.float32) + eb2_ref[...])
        outs[0][...] = m
        if has_coord:
            t = _silu(jnp.dot(m, cw1_ref[...],
                              preferred_element_type=jnp.float32) + cb1_ref[...])
            cw = jnp.tanh(jnp.dot(t, cw2_ref[...],
                                  preferred_element_type=jnp.float32))
            q = cw * lax.rsqrt(dsq + 1e-8)
            qcols = jnp.concatenate(
                [q[r * CH:(r + 1) * CH] for r in range(nrow)], axis=1)
            qpacked = lax.dot_general(qcols, eye, (((0,), (0,)), ((), ())),
                                      preferred_element_type=jnp.float32)
            outs[1][...] = qpacked[None]

    out_specs = [pl.BlockSpec((BE, H), lambda i: (i, 0))]
    out_shape = [jax.ShapeDtypeStruct((e, H), jnp.float32)]
    if has_coord:
        out_specs.append(pl.BlockSpec((1, nrow, CH), lambda i: (i, 0, 0)))
        out_shape.append(jax.ShapeDtypeStruct((e // BE, nrow, CH), jnp.float32))
    res = pl.pallas_call(
        body,
        grid=(e // BE,),
        in_specs=[pl.BlockSpec((BE, H), lambda i: (i, 0)),
                  pl.BlockSpec((1, nrow, CH), lambda i: (i, 0, 0)),
                  pl.BlockSpec((BE, H), lambda i: (i, 0)),
                  _full((1, H)), _full((H, H)), _full((H, H)), _full((1, H)),
                  _full((H, H)), _full((1, H)), _full((H, 1))],
        out_specs=out_specs,
        out_shape=out_shape,
    )(sg, jnp.reshape(dsqp, (e // BE, nrow, CH)), ea,
      w1d, w1e, ew2, eb2, cw1, cb1, cw2)
    if has_coord:
        return res[0], jnp.reshape(res[1], (e // CH, CH))
    return res[0], None


def _tc_xupdate(x3, accg):
    n = x3.shape[1]

    def body(x_ref, g_ref, xn_ref):
        ag = g_ref[...]
        xv = x_ref[...]
        s1 = ag[3:4] + ag[8:9]
        deg = jnp.maximum(ag[4:5] + ag[9:10], 1.0)
        sxc = ag[0:3] + ag[5:8]
        xn_ref[...] = xv + (xv * s1 - sxc) / deg

    return pl.pallas_call(
        body,
        out_shape=jax.ShapeDtypeStruct((3, n), jnp.float32),
    )(x3, accg)


def _tc_node(h, am0, am1, nw1h, nw1m, nb1, nw2, nb2, ln_g, ln_b,
             w1a, w1b, eb1):
    n = h.shape[0]

    def body(h_ref, a0_ref, a1_ref, w1h_ref, w1m_ref, b1_ref,
             w2_ref, b2_ref, lg_ref, lb_ref, wa_ref, wb_ref, eb_ref,
             hn_ref, ta_ref, tb_ref):
        hv = h_ref[...]
        m_i = a0_ref[...] + a1_ref[...]
        pre = (jnp.dot(hv, w1h_ref[...], preferred_element_type=jnp.float32)
               + jnp.dot(m_i, w1m_ref[...], preferred_element_type=jnp.float32)
               + b1_ref[...])
        hu = jnp.dot(_silu(pre), w2_ref[...],
                     preferred_element_type=jnp.float32) + b2_ref[...]
        hn = _ln(hv + hu, lg_ref[...], lb_ref[...])
        hn_ref[...] = hn
        ta_ref[...] = jnp.dot(hn, wa_ref[...],
                              preferred_element_type=jnp.float32) + eb_ref[...]
        tb_ref[...] = jnp.dot(hn, wb_ref[...],
                              preferred_element_type=jnp.float32)

    return pl.pallas_call(
        body,
        grid=(n // BN,),
        in_specs=[pl.BlockSpec((BN, H), lambda i: (i, 0)),
                  pl.BlockSpec((BN, H), lambda i: (i, 0)),
                  pl.BlockSpec((BN, H), lambda i: (i, 0)),
                  _full((H, H)), _full((H, H)), _full((1, H)),
                  _full((H, H)), _full((1, H)), _full((1, H)), _full((1, H)),
                  _full((H, H)), _full((H, H)), _full((1, H))],
        out_specs=[pl.BlockSpec((BN, H), lambda i: (i, 0)),
                   pl.BlockSpec((BN, H), lambda i: (i, 0)),
                   pl.BlockSpec((BN, H), lambda i: (i, 0))],
        out_shape=[jax.ShapeDtypeStruct((n, H), jnp.float32),
                   jax.ShapeDtypeStruct((n, H), jnp.float32),
                   jax.ShapeDtypeStruct((n, H), jnp.float32)],
    )(h, am0, am1, nw1h, nw1m, nb1, nw2, nb2, ln_g, ln_b, w1a, w1b, eb1)


def _tc_node_last(h, am0, am1, nw1h, nw1m, nb1, nw2, nb2, ln_g, ln_b,
                  pw1, pb1, pw2, pb2):
    n = h.shape[0]

    def body(h_ref, a0_ref, a1_ref, w1h_ref, w1m_ref, b1_ref, w2_ref, b2_ref,
             lg_ref, lb_ref, pw1_ref, pb1_ref, pw2_ref, pb2_ref,
             hn_ref, s_ref):
        hv = h_ref[...]
        m_i = a0_ref[...] + a1_ref[...]
        pre = (jnp.dot(hv, w1h_ref[...], preferred_element_type=jnp.float32)
               + jnp.dot(m_i, w1m_ref[...], preferred_element_type=jnp.float32)
               + b1_ref[...])
        hu = jnp.dot(_silu(pre), w2_ref[...],
                     preferred_element_type=jnp.float32) + b2_ref[...]
        hn = _ln(hv + hu, lg_ref[...], lb_ref[...])
        hn_ref[...] = hn
        t = jnp.tanh(jnp.dot(hn, pw1_ref[...],
                             preferred_element_type=jnp.float32) + pb1_ref[...])
        s_ref[...] = jnp.dot(t, pw2_ref[...],
                             preferred_element_type=jnp.float32) + pb2_ref[...]

    return pl.pallas_call(
        body,
        grid=(n // BN,),
        in_specs=[pl.BlockSpec((BN, H), lambda i: (i, 0)),
                  pl.BlockSpec((BN, H), lambda i: (i, 0)),
                  pl.BlockSpec((BN, H), lambda i: (i, 0)),
                  _full((H, H)), _full((H, H)), _full((1, H)),
                  _full((H, H)), _full((1, H)), _full((1, H)), _full((1, H)),
                  _full((H, H)), _full((1, H)), _full((H, 1)), _full((1, 1))],
        out_specs=[pl.BlockSpec((BN, H), lambda i: (i, 0)),
                   pl.BlockSpec((BN, 1), lambda i: (i, 0))],
        out_shape=[jax.ShapeDtypeStruct((n, H), jnp.float32),
                   jax.ShapeDtypeStruct((n, 1), jnp.float32)],
    )(h, am0, am1, nw1h, nw1m, nb1, nw2, nb2, ln_g, ln_b, pw1, pb1, pw2, pb2)


def _tc_pool(h4, sarr, batch2d, nb, cw1, cb1, cw2, cb2, cw3, cb3):
    n = h4.shape[0]

    def body(h_ref, s_ref, b_ref, cw1_ref, cb1_ref, cw2_ref, cb2_ref,
             cw3_ref, cb3_ref, out_ref):
        bv = b_ref[...]
        onehot = (bv == lax.broadcasted_iota(jnp.int32, (1, nb), 1))
        sv = s_ref[...]
        s_b = jnp.broadcast_to(sv, (n, nb))
        smax = jnp.max(jnp.where(onehot, s_b, -1e30), axis=0, keepdims=True)
        mm = jnp.where(onehot, jnp.exp(s_b - jnp.broadcast_to(smax, (n, nb))),
                       0.0)
        ssum = lax.dot_general(mm, jnp.ones((n, 1), jnp.float32),
                               (((0,), (0,)), ((), ())),
                               preferred_element_type=jnp.float32)
        gnum = lax.dot_general(mm, h_ref[...], (((0,), (0,)), ((), ())),
                               preferred_element_type=jnp.float32)
        g = gnum / (ssum + 1e-16)
        inv = 1.0 / jnp.sqrt(jnp.float32(1.0 + 1e-5))
        z = _silu(jnp.dot(g, cw1_ref[...],
                          preferred_element_type=jnp.float32) + cb1_ref[...]) * inv
        z = _silu(jnp.dot(z, cw2_ref[...],
                          preferred_element_type=jnp.float32) + cb2_ref[...]) * inv
        out_ref[...] = jnp.dot(z, cw3_ref[...],
                               preferred_element_type=jnp.float32) + cb3_ref[...]

    return pl.pallas_call(
        body,
        out_shape=jax.ShapeDtypeStruct((nb, 1), jnp.float32),
    )(h4, sarr, batch2d, cw1, cb1, cw2, cb2, cw3, cb3)


def kernel(h, x, edge_index, edge_attr, batch, params):
    n = h.shape[0]
    row = edge_index[0]
    col = edge_index[1]
    x0, x1, x2 = x[:, 0], x[:, 1], x[:, 2]
    p = params
    layers = p["layers"]

    def r2(v):
        return v.reshape(1, -1)

    def split_ew1(lp):
        ew1 = lp["ew1"]
        return ew1[:H], ew1[H:2 * H], ew1[2 * H:2 * H + 1], ew1[2 * H + 1:]

    w1a0, w1b0, _, _ = split_ew1(layers[0])
    hcur, tab_a, tab_b = _enc_node_prep(
        h, p["ne_w"], r2(p["ne_b"]), r2(p["ne_g"]), r2(p["ne_beta"]),
        w1a0, w1b0, r2(layers[0]["eb1"]))
    ea = _enc_edge(edge_attr, p["ee_w"], r2(p["ee_b"]))
    zeros_m = jnp.zeros((n, H), jnp.float32)
    zeros_1 = jnp.zeros((n,), jnp.float32)
    sarr = None
    for i, lp in enumerate(layers):
        _, _, w1d, w1e = split_ew1(lp)
        has_coord = i < len(layers) - 1
        sg, dsqp = _sc_gather(tab_a, tab_b, row, col, x0, x1, x2)
        if has_coord:
            cw1, cb1, cw2 = lp["cw1"], r2(lp["cb1"]), lp["cw2"]
        else:
            cw1 = jnp.zeros((H, H), jnp.float32)
            cb1 = jnp.zeros((1, H), jnp.float32)
            cw2 = jnp.zeros((H, 1), jnp.float32)
        medge, qpk = _tc_edge(sg, dsqp, ea, w1d, w1e, lp["ew2"], r2(lp["eb2"]),
                              cw1, cb1, cw2, has_coord)
        nw1 = lp["nw1"]
        if has_coord:
            acc_m, acc_g = _sc_scatter(medge, row, zeros_m, qpk, col,
                                       x0, x1, x2, zeros_1)
            nlp = layers[i + 1]
            w1a, w1b, _, _ = split_ew1(nlp)
            xn3 = _tc_xupdate(jnp.stack([x0, x1, x2]), acc_g)
            x0, x1, x2 = xn3[0], xn3[1], xn3[2]
            hcur, tab_a, tab_b = _tc_node(
                hcur, acc_m[0], acc_m[1],
                nw1[:H], nw1[H:], r2(lp["nb1"]), lp["nw2"], r2(lp["nb2"]),
                r2(lp["ln_g"]), r2(lp["ln_b"]),
                w1a, w1b, r2(nlp["eb1"]))
        else:
            (acc_m,) = _sc_scatter(medge, row, zeros_m)
            hcur, sarr = _tc_node_last(
                hcur, acc_m[0], acc_m[1],
                nw1[:H], nw1[H:], r2(lp["nb1"]), lp["nw2"], r2(lp["nb2"]),
                r2(lp["ln_g"]), r2(lp["ln_b"]),
                p["pw1"], r2(p["pb1"]), p["pw2"], r2(p["pb2"]))
    nb = 64
    return _tc_pool(hcur, sarr, batch.reshape(n, 1), nb,
                    p["cw1"], r2(p["cb1"]), p["cw2"], r2(p["cb2"]),
                    p["cw3"], r2(p["cb3"]))


# speedup vs baseline: 4.8590x; 1.1062x over previous
import functools

import jax
import jax.numpy as jnp
from jax import lax
from jax.experimental import pallas as pl
from jax.experimental.pallas import tpu as pltpu
from jax.experimental.pallas import tpu_sc as plsc

NC = 2
NS = 16
CH = 128
H = 128
BE = 1280
BN = 2000


def _silu(t):
    return t * jax.nn.sigmoid(t)


def _ln(t, g, b):
    mu = jnp.mean(t, -1, keepdims=True)
    var = jnp.mean((t - mu) ** 2, -1, keepdims=True)
    return (t - mu) * lax.rsqrt(var + 1e-5) * g + b


def _sc_gather(tab_a, tab_b, row, col, x0, x1, x2):
    n_edges = row.shape[0]
    nchunk = n_edges // CH
    per_core = nchunk // NC
    per_tile = per_core // NS
    if per_tile % 2:
        per_tile -= 1
    ngrp = per_tile // 2
    n_extra = per_core - per_tile * NS
    assert 0 <= n_extra < NS
    mesh = plsc.VectorSubcoreMesh(core_axis_name="c", subcore_axis_name="s")

    @functools.partial(
        pl.kernel,
        out_type=[jax.ShapeDtypeStruct((n_edges, H), jnp.float32),
                  jax.ShapeDtypeStruct((nchunk, CH), jnp.float32)],
        mesh=mesh,
        scratch_types=[
            pltpu.VMEM((2, CH), jnp.int32),
            pltpu.VMEM((2, CH), jnp.int32),
            pltpu.VMEM((2, CH, H), jnp.float32),
            pltpu.VMEM((2, CH), jnp.float32),
            pltpu.VMEM((2, CH), jnp.float32),
            pltpu.VMEM((2, CH), jnp.float32),
            pltpu.VMEM((2, CH), jnp.float32),
            pltpu.VMEM((2, CH), jnp.float32),
            pltpu.VMEM((2, CH), jnp.float32),
            pltpu.VMEM((2, CH), jnp.float32),
        ] + [pltpu.SemaphoreType.DMA] * 10,
    )
    def k(ta, tb, row_h, col_h, x0_h, x1_h, x2_h, sg_h, dsq_h,
          idxr, idxc, buf, xr0, xr1, xr2, xc0, xc1, xc2, dsqv,
          sl0, sl1, sa0, sa1, sb0, sb1, sx0, sx1, sw0, sw1):
        c = lax.axis_index("c")
        s = lax.axis_index("s")
        start = c * per_core + s * per_tile
        sl = (sl0, sl1)
        sa = (sa0, sa1)
        sb = (sb0, sb1)
        sx = (sx0, sx1)
        sw = (sw0, sw1)

        def base_of(t):
            return pl.multiple_of(t * CH, CH)

        def grp(g, carry):
            bases = [base_of(start + g * 2 + b) for b in (0, 1)]
            @pl.when(g > 0)
            def _():
                for b in (0, 1):
                    pltpu.make_async_copy(buf.at[b], sg_h.at[pl.ds(0, CH)],
                                          sw[b]).wait()
                    pltpu.make_async_copy(dsqv.at[b], dsq_h.at[0],
                                          sw[b]).wait()
            ls = []
            for b in (0, 1):
                ls.append(pltpu.async_copy(row_h.at[pl.ds(bases[b], CH)],
                                           idxr.at[b], sl[b]))
                ls.append(pltpu.async_copy(col_h.at[pl.ds(bases[b], CH)],
                                           idxc.at[b], sl[b]))
            cas, xs = [], []
            for b in (0, 1):
                ls[2 * b].wait()
                ls[2 * b + 1].wait()
                cas.append(pltpu.async_copy(ta.at[idxr.at[b]], buf.at[b],
                                            sa[b]))
                xs.append((pltpu.async_copy(x0_h.at[idxr.at[b]], xr0.at[b], sx[b]),
                           pltpu.async_copy(x1_h.at[idxr.at[b]], xr1.at[b], sx[b]),
                           pltpu.async_copy(x2_h.at[idxr.at[b]], xr2.at[b], sx[b]),
                           pltpu.async_copy(x0_h.at[idxc.at[b]], xc0.at[b], sx[b]),
                           pltpu.async_copy(x1_h.at[idxc.at[b]], xc1.at[b], sx[b]),
                           pltpu.async_copy(x2_h.at[idxc.at[b]], xc2.at[b], sx[b])))
            cbs = []
            for b in (0, 1):
                cas[b].wait()
                cbs.append(pltpu.async_copy(tb.at[idxc.at[b]], buf.at[b],
                                            sb[b], add=True))
            for b in (0, 1):
                for xcp in xs[b]:
                    xcp.wait()
                for g16 in range(CH // 16):
                    d = pl.ds(g16 * 16, 16)
                    a = xr0[b, d] - xc0[b, d]
                    bb = xr1[b, d] - xc1[b, d]
                    cc = xr2[b, d] - xc2[b, d]
                    dsqv[b, d] = a * a + bb * bb + cc * cc
            for b in (0, 1):
                cbs[b].wait()
                pltpu.async_copy(buf.at[b], sg_h.at[pl.ds(bases[b], CH)], sw[b])
                pltpu.async_copy(dsqv.at[b], dsq_h.at[start + g * 2 + b], sw[b])
            return carry

        lax.fori_loop(0, ngrp, grp, 0)
        for b in (0, 1):
            pltpu.make_async_copy(buf.at[b], sg_h.at[pl.ds(0, CH)], sw[b]).wait()
            pltpu.make_async_copy(dsqv.at[b], dsq_h.at[0], sw[b]).wait()

        @pl.when(s < n_extra)
        def _():
            t = c * per_core + NS * per_tile + s
            base = base_of(t)
            pltpu.sync_copy(row_h.at[pl.ds(base, CH)], idxr.at[0])
            pltpu.sync_copy(col_h.at[pl.ds(base, CH)], idxc.at[0])
            ca = pltpu.async_copy(ta.at[idxr.at[0]], buf.at[0], sa0)
            g0 = pltpu.async_copy(x0_h.at[idxr.at[0]], xr0.at[0], sx0)
            g1 = pltpu.async_copy(x1_h.at[idxr.at[0]], xr1.at[0], sx0)
            g2 = pltpu.async_copy(x2_h.at[idxr.at[0]], xr2.at[0], sx0)
            g3 = pltpu.async_copy(x0_h.at[idxc.at[0]], xc0.at[0], sx0)
            g4 = pltpu.async_copy(x1_h.at[idxc.at[0]], xc1.at[0], sx0)
            g5 = pltpu.async_copy(x2_h.at[idxc.at[0]], xc2.at[0], sx0)
            ca.wait()
            cb = pltpu.async_copy(tb.at[idxc.at[0]], buf.at[0], sb0, add=True)
            g0.wait(); g1.wait(); g2.wait()
            g3.wait(); g4.wait(); g5.wait()
            for g16 in range(CH // 16):
                d = pl.ds(g16 * 16, 16)
                a = xr0[0, d] - xc0[0, d]
                bb = xr1[0, d] - xc1[0, d]
                cc = xr2[0, d] - xc2[0, d]
                dsqv[0, d] = a * a + bb * bb + cc * cc
            cb.wait()
            pltpu.sync_copy(buf.at[0], sg_h.at[pl.ds(base, CH)])
            pltpu.sync_copy(dsqv.at[0], dsq_h.at[t])

    return k(tab_a, tab_b, row, col, x0, x1, x2)


def _sc_scatter(medge, row, zeros_m, qpk=None, col=None, x0=None, x1=None,
                x2=None, zeros_1=None):
    has_coord = qpk is not None
    n_edges = row.shape[0]
    n_nodes = zeros_m.shape[0]
    nchunk = n_edges // CH
    per_core = nchunk // NC
    kmax = (per_core + NS - 1) // NS
    rpt_a = -(-n_nodes // NS) + 7 & ~7
    rpt_lo = rpt_a * (NS - 1)
    rpt_b = n_nodes - rpt_lo
    mesh = plsc.VectorSubcoreMesh(core_axis_name="c", subcore_axis_name="s")

    per_tile = per_core // NS
    if per_tile % 2:
        per_tile -= 1
    ngrp = per_tile // 2
    n_extra = per_core - per_tile * NS
    assert 0 <= n_extra < NS
    out_type = [jax.ShapeDtypeStruct((NC, n_nodes, H), jnp.float32)]
    scratch = [
        pltpu.VMEM((2, CH), jnp.int32),
        pltpu.VMEM((2, CH, H), jnp.float32),
        pltpu.VMEM_SHARED((n_nodes, H), jnp.float32),
    ]
    if has_coord:
        out_type += [jax.ShapeDtypeStruct((n_nodes,), jnp.float32)] * 10
        scratch += [
            pltpu.VMEM((2, CH), jnp.int32),
            pltpu.VMEM((2, CH), jnp.float32),
            pltpu.VMEM((2, CH), jnp.float32),
            pltpu.VMEM((2, CH), jnp.float32),
            pltpu.VMEM((2, CH), jnp.float32),
            pltpu.VMEM((2, CH), jnp.float32),
            pltpu.VMEM((2, CH), jnp.float32),
            pltpu.VMEM((2, CH), jnp.float32),
            pltpu.VMEM((CH,), jnp.float32),
            pltpu.VMEM_SHARED((n_nodes,), jnp.float32),
            pltpu.VMEM_SHARED((n_nodes,), jnp.float32),
            pltpu.VMEM_SHARED((n_nodes,), jnp.float32),
            pltpu.VMEM_SHARED((n_nodes,), jnp.float32),
            pltpu.VMEM_SHARED((n_nodes,), jnp.float32),
        ]
    scratch += [pltpu.SemaphoreType.DMA] * 6

    @functools.partial(pl.kernel, out_type=out_type, mesh=mesh,
                       scratch_types=scratch)
    def k(*refs):
        if has_coord:
            (m_h, row_h, z_h, qpk_h, col_h, x0_h, x1_h, x2_h, z1_h,
             accm_out, g0, g1, g2, g3, g4, g5, g6, g7, g8, g9,
             idx, mbuf, accm,
             idxc, qv, xc0, xc1, xc2, qx0, qx1, qx2, ones,
             aq0, aq1, aq2, aq3, adeg,
             sml0, sml1, sxg0, sxg1, ssa0, ssa1) = refs
        else:
            (m_h, row_h, z_h, accm_out, idx, mbuf, accm,
             sml0, sml1, sxg0, sxg1, ssa0, ssa1) = refs
        c = lax.axis_index("c")
        s = lax.axis_index("s")
        start = c * per_core + s * per_tile
        sml = (sml0, sml1)
        sxg = (sxg0, sxg1)
        ssa = (ssa0, ssa1)
        off = pl.multiple_of(s * rpt_a, 8)

        @pl.when(s < NS - 1)
        def _():
            pltpu.sync_copy(z_h.at[pl.ds(off, rpt_a)],
                            accm.at[pl.ds(off, rpt_a)])

        @pl.when(s == NS - 1)
        def _():
            pltpu.sync_copy(z_h.at[pl.ds(rpt_lo, rpt_b)],
                            accm.at[pl.ds(rpt_lo, rpt_b)])
        if has_coord:
            for g in range(CH // 16):
                ones[pl.ds(g * 16, 16)] = jnp.full((16,), 1.0, jnp.float32)

            @pl.when(s == 0)
            def _():
                for a in (aq0, aq1, aq2, aq3, adeg):
                    pltpu.sync_copy(z1_h, a)
        plsc.subcore_barrier()

        def drain(b):
            pltpu.make_async_copy(mbuf.at[b], accm.at[pl.ds(0, CH)],
                                  ssa[b]).wait()
            if has_coord:
                for src, dst in ((qx0, aq0), (qx1, aq1), (qx2, aq2),
                                 (qv, aq3), (qv, adeg)):
                    pltpu.make_async_copy(src.at[b], dst.at[pl.ds(0, CH)],
                                          ssa[b]).wait()

        def do_chunk_loads(t, b):
            base = pl.multiple_of(t * CH, CH)
            lds = [pltpu.async_copy(row_h.at[pl.ds(base, CH)], idx.at[b],
                                    sml[b]),
                   pltpu.async_copy(m_h.at[pl.ds(base, CH)], mbuf.at[b],
                                    sml[b])]
            if has_coord:
                lds.append(pltpu.async_copy(col_h.at[pl.ds(base, CH)],
                                            idxc.at[b], sml[b]))
                lds.append(pltpu.async_copy(qpk_h.at[t], qv.at[b], sml[b]))
            return lds

        def grp(g, carry):
            @pl.when(g > 0)
            def _():
                for b in (0, 1):
                    drain(b)
            lds = [do_chunk_loads(start + g * 2 + b, b) for b in (0, 1)]
            xg = []
            for b in (0, 1):
                for ld in lds[b]:
                    ld.wait()
                pltpu.async_copy(mbuf.at[b], accm.at[idx.at[b]], ssa[b],
                                 add=True)
                if has_coord:
                    xg.append((
                        pltpu.async_copy(x0_h.at[idxc.at[b]], xc0.at[b], sxg[b]),
                        pltpu.async_copy(x1_h.at[idxc.at[b]], xc1.at[b], sxg[b]),
                        pltpu.async_copy(x2_h.at[idxc.at[b]], xc2.at[b], sxg[b])))
            if has_coord:
                for b in (0, 1):
                    for xcp in xg[b]:
                        xcp.wait()
                    for g16 in range(CH // 16):
                        d = pl.ds(g16 * 16, 16)
                        q = qv[b, d]
                        qx0[b, d] = q * xc0[b, d]
                        qx1[b, d] = q * xc1[b, d]
                        qx2[b, d] = q * xc2[b, d]
                    pltpu.async_copy(qx0.at[b], aq0.at[idx.at[b]], ssa[b],
                                     add=True)
                    pltpu.async_copy(qx1.at[b], aq1.at[idx.at[b]], ssa[b],
                                     add=True)
                    pltpu.async_copy(qx2.at[b], aq2.at[idx.at[b]], ssa[b],
                                     add=True)
                    pltpu.async_copy(qv.at[b], aq3.at[idx.at[b]], ssa[b],
                                     add=True)
                    pltpu.async_copy(ones, adeg.at[idx.at[b]], ssa[b],
                                     add=True)
            return carry

        lax.fori_loop(0, ngrp, grp, 0)
        for b in (0, 1):
            drain(b)

        @pl.when(s < n_extra)
        def _():
            t = c * per_core + NS * per_tile + s
            base = pl.multiple_of(t * CH, CH)
            pltpu.sync_copy(row_h.at[pl.ds(base, CH)], idx.at[0])
            pltpu.sync_copy(m_h.at[pl.ds(base, CH)], mbuf.at[0])
            pltpu.sync_copy(mbuf.at[0], accm.at[idx.at[0]], add=True)
            if has_coord:
                pltpu.sync_copy(col_h.at[pl.ds(base, CH)], idxc.at[0])
                pltpu.sync_copy(qpk_h.at[t], qv.at[0])
                c0 = pltpu.async_copy(x0_h.at[idxc.at[0]], xc0.at[0], sxg0)
                c1 = pltpu.async_copy(x1_h.at[idxc.at[0]], xc1.at[0], sxg0)
                c2 = pltpu.async_copy(x2_h.at[idxc.at[0]], xc2.at[0], sxg0)
                c0.wait(); c1.wait(); c2.wait()
                for g16 in range(CH // 16):
                    d = pl.ds(g16 * 16, 16)
                    q = qv[0, d]
                    qx0[0, d] = q * xc0[0, d]
                    qx1[0, d] = q * xc1[0, d]
                    qx2[0, d] = q * xc2[0, d]
                pltpu.sync_copy(qx0.at[0], aq0.at[idx.at[0]], add=True)
                pltpu.sync_copy(qx1.at[0], aq1.at[idx.at[0]], add=True)
                pltpu.sync_copy(qx2.at[0], aq2.at[idx.at[0]], add=True)
                pltpu.sync_copy(qv.at[0], aq3.at[idx.at[0]], add=True)
                pltpu.sync_copy(ones, adeg.at[idx.at[0]], add=True)
        plsc.subcore_barrier()

        @pl.when(s < NS - 1)
        def _():
            pltpu.sync_copy(accm.at[pl.ds(off, rpt_a)],
                            accm_out.at[c, pl.ds(off, rpt_a)])

        @pl.when(s == NS - 1)
        def _():
            pltpu.sync_copy(accm.at[pl.ds(rpt_lo, rpt_b)],
                            accm_out.at[c, pl.ds(rpt_lo, rpt_b)])
        if has_coord:
            @pl.when((s == 1) & (c == 0))
            def _():
                for dst, a in zip((g0, g1, g2, g3, g4),
                                  (aq0, aq1, aq2, aq3, adeg)):
                    pltpu.sync_copy(a, dst)

            @pl.when((s == 1) & (c == 1))
            def _():
                for dst, a in zip((g5, g6, g7, g8, g9),
                                  (aq0, aq1, aq2, aq3, adeg)):
                    pltpu.sync_copy(a, dst)

    if has_coord:
        res = k(medge, row, zeros_m, qpk, col, x0, x1, x2, zeros_1)
        return res[0], jnp.stack(res[1:])
    return k(medge, row, zeros_m)


def _full(shape):
    nd = len(shape)
    return pl.BlockSpec(shape, lambda i: (0,) * nd)


def _enc_node_prep(h_in, ne_w, ne_b, ne_g, ne_beta, w1a, w1b, eb1):
    n = h_in.shape[0]
    nf = h_in.shape[1]

    def body(h_ref, w_ref, b_ref, g_ref, beta_ref, wa_ref, wb_ref, eb_ref,
             h0_ref, a_ref, b2_ref):
        h0 = _ln(_silu(jnp.dot(h_ref[...], w_ref[...],
                               preferred_element_type=jnp.float32) + b_ref[...]),
                 g_ref[...], beta_ref[...])
        h0_ref[...] = h0
        a_ref[...] = jnp.dot(h0, wa_ref[...],
                             preferred_element_type=jnp.float32) + eb_ref[...]
        b2_ref[...] = jnp.dot(h0, wb_ref[...],
                              preferred_element_type=jnp.float32)

    return pl.pallas_call(
        body,
        grid=(n // BN,),
        in_specs=[pl.BlockSpec((BN, nf), lambda i: (i, 0)),
                  _full((nf, H)), _full((1, H)), _full((1, H)), _full((1, H)),
                  _full((H, H)), _full((H, H)), _full((1, H))],
        out_specs=[pl.BlockSpec((BN, H), lambda i: (i, 0))] * 3,
        out_shape=[jax.ShapeDtypeStruct((n, H), jnp.float32)] * 3,
    )(h_in, ne_w, ne_b, ne_g, ne_beta, w1a, w1b, eb1)


def _enc_edge(edge_attr, ee_w, ee_b):
    e = edge_attr.shape[0]
    ef = edge_attr.shape[1]

    def body(ea_ref, w_ref, b_ref, out_ref):
        out_ref[...] = _silu(jnp.dot(ea_ref[...], w_ref[...],
                                     preferred_element_type=jnp.float32)
                             + b_ref[...])

    return pl.pallas_call(
        body,
        grid=(e // BE,),
        in_specs=[pl.BlockSpec((BE, ef), lambda i: (i, 0)),
                  _full((ef, H)), _full((1, H))],
        out_specs=pl.BlockSpec((BE, H), lambda i: (i, 0)),
        out_shape=jax.ShapeDtypeStruct((e, H), jnp.float32),
    )(edge_attr, ee_w, ee_b)


def _tc_edge(sg, dsqp, ea, w1d, w1e, ew2, eb2, cw1, cb1, cw2, has_coord):
    e = sg.shape[0]
    nrow = BE // CH

    def body(sg_ref, dsq_ref, ea_ref, w1d_ref, w1e_ref, ew2_ref, eb2_ref,
             cw1_ref, cb1_ref, cw2_ref, *outs):
        eye = jnp.eye(CH, dtype=jnp.float32)
        packed = dsq_ref[...][0]
        cols = lax.dot_general(eye, packed, (((1,), (1,)), ((), ())),
                               preferred_element_type=jnp.float32)
        dsq = jnp.concatenate([cols[:, r:r + 1] for r in range(nrow)],
                              axis=0)
        pre = (sg_ref[...] + dsq * w1d_ref[...]
               + jnp.dot(ea_ref[...], w1e_ref[...],
                         preferred_element_type=jnp.float32))
        m1 = _silu(pre)
        m = _silu(jnp.dot(m1, ew2_ref[...],
                          preferred_element_type=jnp.float32) + eb2_ref[...])
        outs[0][...] = m
        if has_coord:
            t = _silu(jnp.dot(m, cw1_ref[...],
                              preferred_element_type=jnp.float32) + cb1_ref[...])
            cw = jnp.tanh(jnp.dot(t, cw2_ref[...],
                                  preferred_element_type=jnp.float32))
            q = cw * lax.rsqrt(dsq + 1e-8)
            qcols = jnp.concatenate(
                [q[r * CH:(r + 1) * CH] for r in range(nrow)], axis=1)
            qpacked = lax.dot_general(qcols, eye, (((0,), (0,)), ((), ())),
                                      preferred_element_type=jnp.float32)
            outs[1][...] = qpacked[None]

    out_specs = [pl.BlockSpec((BE, H), lambda i: (i, 0))]
    out_shape = [jax.ShapeDtypeStruct((e, H), jnp.float32)]
    if has_coord:
        out_specs.append(pl.BlockSpec((1, nrow, CH), lambda i: (i, 0, 0)))
        out_shape.append(jax.ShapeDtypeStruct((e // BE, nrow, CH), jnp.float32))
    res = pl.pallas_call(
        body,
        grid=(e // BE,),
        in_specs=[pl.BlockSpec((BE, H), lambda i: (i, 0)),
                  pl.BlockSpec((1, nrow, CH), lambda i: (i, 0, 0)),
                  pl.BlockSpec((BE, H), lambda i: (i, 0)),
                  _full((1, H)), _full((H, H)), _full((H, H)), _full((1, H)),
                  _full((H, H)), _full((1, H)), _full((H, 1))],
        out_specs=out_specs,
        out_shape=out_shape,
    )(sg, jnp.reshape(dsqp, (e // BE, nrow, CH)), ea,
      w1d, w1e, ew2, eb2, cw1, cb1, cw2)
    if has_coord:
        return res[0], jnp.reshape(res[1], (e // CH, CH))
    return res[0], None


def _tc_xupdate(x3, accg):
    n = x3.shape[1]
    nparts = accg.shape[0] // 5

    def body(x_ref, g_ref, xn_ref):
        ag = g_ref[...]
        xv = x_ref[...]
        s1 = sum(ag[5 * p + 3:5 * p + 4] for p in range(nparts))
        deg = jnp.maximum(
            sum(ag[5 * p + 4:5 * p + 5] for p in range(nparts)), 1.0)
        sxc = sum(ag[5 * p:5 * p + 3] for p in range(nparts))
        xn_ref[...] = xv + (xv * s1 - sxc) / deg

    return pl.pallas_call(
        body,
        out_shape=jax.ShapeDtypeStruct((3, n), jnp.float32),
    )(x3, accg)


def _tc_node(h, ams, nw1h, nw1m, nb1, nw2, nb2, ln_g, ln_b,
             w1a, w1b, eb1):
    n = h.shape[0]
    npart = len(ams)

    def body(h_ref, *refs):
        (a_refs, (w1h_ref, w1m_ref, b1_ref, w2_ref, b2_ref, lg_ref, lb_ref,
                  wa_ref, wb_ref, eb_ref),
         (hn_ref, ta_ref, tb_ref)) = (refs[:npart], refs[npart:npart + 10],
                                      refs[npart + 10:])
        hv = h_ref[...]
        m_i = sum(a[...] for a in a_refs)
        pre = (jnp.dot(hv, w1h_ref[...], preferred_element_type=jnp.float32)
               + jnp.dot(m_i, w1m_ref[...], preferred_element_type=jnp.float32)
               + b1_ref[...])
        hu = jnp.dot(_silu(pre), w2_ref[...],
                     preferred_element_type=jnp.float32) + b2_ref[...]
        hn = _ln(hv + hu, lg_ref[...], lb_ref[...])
        hn_ref[...] = hn
        ta_ref[...] = jnp.dot(hn, wa_ref[...],
                              preferred_element_type=jnp.float32) + eb_ref[...]
        tb_ref[...] = jnp.dot(hn, wb_ref[...],
                              preferred_element_type=jnp.float32)

    return pl.pallas_call(
        body,
        grid=(n // BN,),
        in_specs=[pl.BlockSpec((BN, H), lambda i: (i, 0))] * (1 + npart)
                 + [_full((H, H)), _full((H, H)), _full((1, H)),
                    _full((H, H)), _full((1, H)), _full((1, H)), _full((1, H)),
                    _full((H, H)), _full((H, H)), _full((1, H))],
        out_specs=[pl.BlockSpec((BN, H), lambda i: (i, 0)),
                   pl.BlockSpec((BN, H), lambda i: (i, 0)),
                   pl.BlockSpec((BN, H), lambda i: (i, 0))],
        out_shape=[jax.ShapeDtypeStruct((n, H), jnp.float32),
                   jax.ShapeDtypeStruct((n, H), jnp.float32),
                   jax.ShapeDtypeStruct((n, H), jnp.float32)],
    )(h, *ams, nw1h, nw1m, nb1, nw2, nb2, ln_g, ln_b, w1a, w1b, eb1)


def _tc_node_last(h, ams, nw1h, nw1m, nb1, nw2, nb2, ln_g, ln_b,
                  pw1, pb1, pw2, pb2):
    n = h.shape[0]
    npart = len(ams)

    def body(h_ref, *refs):
        (a_refs, (w1h_ref, w1m_ref, b1_ref, w2_ref, b2_ref,
                  lg_ref, lb_ref, pw1_ref, pb1_ref, pw2_ref, pb2_ref),
         (hn_ref, s_ref)) = (refs[:npart], refs[npart:npart + 11],
                             refs[npart + 11:])
        hv = h_ref[...]
        m_i = sum(a[...] for a in a_refs)
        pre = (jnp.dot(hv, w1h_ref[...], preferred_element_type=jnp.float32)
               + jnp.dot(m_i, w1m_ref[...], preferred_element_type=jnp.float32)
               + b1_ref[...])
        hu = jnp.dot(_silu(pre), w2_ref[...],
                     preferred_element_type=jnp.float32) + b2_ref[...]
        hn = _ln(hv + hu, lg_ref[...], lb_ref[...])
        hn_ref[...] = hn
        t = jnp.tanh(jnp.dot(hn, pw1_ref[...],
                             preferred_element_type=jnp.float32) + pb1_ref[...])
        s_ref[...] = jnp.dot(t, pw2_ref[...],
                             preferred_element_type=jnp.float32) + pb2_ref[...]

    return pl.pallas_call(
        body,
        grid=(n // BN,),
        in_specs=[pl.BlockSpec((BN, H), lambda i: (i, 0))] * (1 + npart)
                 + [_full((H, H)), _full((H, H)), _full((1, H)),
                    _full((H, H)), _full((1, H)), _full((1, H)), _full((1, H)),
                    _full((H, H)), _full((1, H)), _full((H, 1)), _full((1, 1))],
        out_specs=[pl.BlockSpec((BN, H), lambda i: (i, 0)),
                   pl.BlockSpec((BN, 1), lambda i: (i, 0))],
        out_shape=[jax.ShapeDtypeStruct((n, H), jnp.float32),
                   jax.ShapeDtypeStruct((n, 1), jnp.float32)],
    )(h, *ams, nw1h, nw1m, nb1, nw2, nb2, ln_g, ln_b, pw1, pb1, pw2, pb2)


def _tc_pool(h4, sarr, batch2d, nb, cw1, cb1, cw2, cb2, cw3, cb3):
    n = h4.shape[0]

    def body(h_ref, s_ref, b_ref, cw1_ref, cb1_ref, cw2_ref, cb2_ref,
             cw3_ref, cb3_ref, out_ref):
        bv = b_ref[...]
        onehot = (bv == lax.broadcasted_iota(jnp.int32, (1, nb), 1))
        sv = s_ref[...]
        s_b = jnp.broadcast_to(sv, (n, nb))
        smax = jnp.max(jnp.where(onehot, s_b, -1e30), axis=0, keepdims=True)
        mm = jnp.where(onehot, jnp.exp(s_b - jnp.broadcast_to(smax, (n, nb))),
                       0.0)
        ssum = lax.dot_general(mm, jnp.ones((n, 1), jnp.float32),
                               (((0,), (0,)), ((), ())),
                               preferred_element_type=jnp.float32)
        gnum = lax.dot_general(mm, h_ref[...], (((0,), (0,)), ((), ())),
                               preferred_element_type=jnp.float32)
        g = gnum / (ssum + 1e-16)
        inv = 1.0 / jnp.sqrt(jnp.float32(1.0 + 1e-5))
        z = _silu(jnp.dot(g, cw1_ref[...],
                          preferred_element_type=jnp.float32) + cb1_ref[...]) * inv
        z = _silu(jnp.dot(z, cw2_ref[...],
                          preferred_element_type=jnp.float32) + cb2_ref[...]) * inv
        out_ref[...] = jnp.dot(z, cw3_ref[...],
                               preferred_element_type=jnp.float32) + cb3_ref[...]

    return pl.pallas_call(
        body,
        out_shape=jax.ShapeDtypeStruct((nb, 1), jnp.float32),
    )(h4, sarr, batch2d, cw1, cb1, cw2, cb2, cw3, cb3)


def kernel(h, x, edge_index, edge_attr, batch, params):
    n = h.shape[0]
    row = edge_index[0]
    col = edge_index[1]
    x0, x1, x2 = x[:, 0], x[:, 1], x[:, 2]
    p = params
    layers = p["layers"]

    def r2(v):
        return v.reshape(1, -1)

    def split_ew1(lp):
        ew1 = lp["ew1"]
        return ew1[:H], ew1[H:2 * H], ew1[2 * H:2 * H + 1], ew1[2 * H + 1:]

    w1a0, w1b0, _, _ = split_ew1(layers[0])
    hcur, tab_a, tab_b = _enc_node_prep(
        h, p["ne_w"], r2(p["ne_b"]), r2(p["ne_g"]), r2(p["ne_beta"]),
        w1a0, w1b0, r2(layers[0]["eb1"]))
    e_total = row.shape[0]

    def _ok(e):
        return e > 0 and e % BE == 0 and ((e // (CH * NC)) // NS) % 2 == 0

    cut = next(k * BE for k in range(e_total // (2 * BE), 0, -1)
               if _ok(k * BE) and _ok(e_total - k * BE))
    halves = [(0, cut), (cut, e_total)]
    rows = [row[a:b] for a, b in halves]
    cols = [col[a:b] for a, b in halves]
    eas = [_enc_edge(edge_attr[a:b], p["ee_w"], r2(p["ee_b"]))
           for a, b in halves]
    zeros_m = jnp.zeros((n, H), jnp.float32)
    zeros_1 = jnp.zeros((n,), jnp.float32)
    sarr = None
    for i, lp in enumerate(layers):
        _, _, w1d, w1e = split_ew1(lp)
        has_coord = i < len(layers) - 1
        if has_coord:
            cw1, cb1, cw2 = lp["cw1"], r2(lp["cb1"]), lp["cw2"]
        else:
            cw1 = jnp.zeros((H, H), jnp.float32)
            cb1 = jnp.zeros((1, H), jnp.float32)
            cw2 = jnp.zeros((H, 1), jnp.float32)
        sgs = [None, None]
        sgs[0] = _sc_gather(tab_a, tab_b, rows[0], cols[0], x0, x1, x2)
        medges, qpks, accs = [None, None], [None, None], [None, None]
        for hf in (0, 1):
            if hf == 0:
                sgs[1] = _sc_gather(tab_a, tab_b, rows[1], cols[1],
                                    x0, x1, x2)
            medges[hf], qpks[hf] = _tc_edge(
                sgs[hf][0], sgs[hf][1], eas[hf], w1d, w1e, lp["ew2"],
                r2(lp["eb2"]), cw1, cb1, cw2, has_coord)
            if has_coord:
                accs[hf] = _sc_scatter(medges[hf], rows[hf], zeros_m,
                                       qpks[hf], cols[hf], x0, x1, x2,
                                       zeros_1)
            else:
                accs[hf] = _sc_scatter(medges[hf], rows[hf], zeros_m)
        nw1 = lp["nw1"]
        ams = [accs[0][0][0], accs[0][0][1], accs[1][0][0], accs[1][0][1]]
        if has_coord:
            nlp = layers[i + 1]
            w1a, w1b, _, _ = split_ew1(nlp)
            accg = jnp.concatenate([accs[0][1], accs[1][1]])
            xn3 = _tc_xupdate(jnp.stack([x0, x1, x2]), accg)
            x0, x1, x2 = xn3[0], xn3[1], xn3[2]
            hcur, tab_a, tab_b = _tc_node(
                hcur, ams,
                nw1[:H], nw1[H:], r2(lp["nb1"]), lp["nw2"], r2(lp["nb2"]),
                r2(lp["ln_g"]), r2(lp["ln_b"]),
                w1a, w1b, r2(nlp["eb1"]))
        else:
            hcur, sarr = _tc_node_last(
                hcur, ams,
                nw1[:H], nw1[H:], r2(lp["nb1"]), lp["nw2"], r2(lp["nb2"]),
                r2(lp["ln_g"]), r2(lp["ln_b"]),
                p["pw1"], r2(p["pb1"]), p["pw2"], r2(p["pb2"]))
    nb = 64
    return _tc_pool(hcur, sarr, batch.reshape(n, 1), nb,
                    p["cw1"], r2(p["cb1"]), p["cw2"], r2(p["cb2"]),
                    p["cw3"], r2(p["cb3"]))
